# Initial kernel scaffold; baseline (speedup 1.0000x reference)
import functools

import jax
import jax.numpy as jnp
from jax import lax
from jax.experimental import pallas as pl
from jax.experimental.pallas import tpu as pltpu
from jax.experimental.pallas import tpu_sc as plsc

_CHUNK = 1600


@functools.lru_cache(maxsize=None)
def _make_gather(B, D):
    info = plsc.get_sparse_core_info()
    nc, ns = info.num_cores, info.num_subcores
    nw = nc * ns
    b_per_w = B // nw
    n_chunks = b_per_w // _CHUNK
    mesh = plsc.VectorSubcoreMesh(core_axis_name="c", subcore_axis_name="s")

    @functools.partial(
        pl.kernel,
        mesh=mesh,
        out_type=jax.ShapeDtypeStruct((B, D), jnp.float32),
        scratch_types=[
            pltpu.VMEM((_CHUNK,), jnp.int32),
            pltpu.VMEM((_CHUNK, D), jnp.float32),
            pltpu.SemaphoreType.DMA,
        ],
    )
    def gather_kernel(idx_hbm, table_hbm, out_hbm, idx_v, rows_v, sem):
        wid = lax.axis_index("s") * nc + lax.axis_index("c")
        base = wid * b_per_w

        def body(g, carry):
            off = base + g * _CHUNK
            pltpu.sync_copy(idx_hbm.at[pl.ds(off, _CHUNK)], idx_v)
            pltpu.async_copy(table_hbm.at[idx_v], rows_v, sem).wait()
            pltpu.sync_copy(rows_v, out_hbm.at[pl.ds(off, _CHUNK)])
            return carry

        lax.fori_loop(0, n_chunks, body, 0)

    return gather_kernel


def kernel(indices, weight):
    n, s = indices.shape
    d = weight.shape[1]
    flat_idx = indices.reshape(n * s).astype(jnp.int32)
    out = _make_gather(n * s, d)(flat_idx, weight)
    return out.reshape(n, s, d)


# speedup vs baseline: 1.1031x; 1.1031x over previous
import functools

import jax
import jax.numpy as jnp
from jax import lax
from jax.experimental import pallas as pl
from jax.experimental.pallas import tpu as pltpu
from jax.experimental.pallas import tpu_sc as plsc

_CHUNK = 1600


@functools.lru_cache(maxsize=None)
def _make_gather(B, D):
    info = plsc.get_sparse_core_info()
    nc, ns = info.num_cores, info.num_subcores
    nw = nc * ns
    b_per_w = B // nw
    n_chunks = b_per_w // _CHUNK
    mesh = plsc.VectorSubcoreMesh(core_axis_name="c", subcore_axis_name="s")

    @functools.partial(
        pl.kernel,
        mesh=mesh,
        out_type=jax.ShapeDtypeStruct((B, D), jnp.float32),
        scratch_types=[
            pltpu.VMEM((_CHUNK,), jnp.int32),
            pltpu.VMEM((_CHUNK, D), jnp.float32),
            pltpu.SemaphoreType.DMA,
        ],
        compiler_params=pltpu.CompilerParams(use_tc_tiling_on_sc=False),
    )
    def gather_kernel(idx_hbm, table_hbm, out_hbm, idx_v, rows_v, sem):
        wid = lax.axis_index("s") * nc + lax.axis_index("c")
        base = wid * b_per_w

        def body(g, carry):
            off = base + g * _CHUNK
            pltpu.sync_copy(idx_hbm.at[pl.ds(off, _CHUNK)], idx_v)
            pltpu.async_copy(table_hbm.at[idx_v], rows_v, sem).wait()
            pltpu.sync_copy(rows_v, out_hbm.at[pl.ds(off, _CHUNK)])
            return carry

        lax.fori_loop(0, n_chunks, body, 0)

    return gather_kernel


def kernel(indices, weight):
    n, s = indices.shape
    d = weight.shape[1]
    flat_idx = indices.reshape(n * s).astype(jnp.int32)
    out = _make_gather(n * s, d)(flat_idx, weight)
    return out.reshape(n, s, d)


# speedup vs baseline: 1.1081x; 1.0046x over previous
import functools

import jax
import jax.numpy as jnp
from jax import lax
from jax.experimental import pallas as pl
from jax.experimental.pallas import tpu as pltpu
from jax.experimental.pallas import tpu_sc as plsc

_CHUNK = 1600
_NBUF = 2


@functools.lru_cache(maxsize=None)
def _make_gather(B, D):
    info = plsc.get_sparse_core_info()
    nc, ns = info.num_cores, info.num_subcores
    nw = nc * ns
    b_per_w = B // nw
    n_chunks = b_per_w // _CHUNK
    assert n_chunks % _NBUF == 0
    mesh = plsc.VectorSubcoreMesh(core_axis_name="c", subcore_axis_name="s")

    scratch = []
    for _ in range(_NBUF):
        scratch += [
            pltpu.VMEM((_CHUNK,), jnp.int32),
            pltpu.VMEM((_CHUNK, D), jnp.float32),
            pltpu.SemaphoreType.DMA,
            pltpu.SemaphoreType.DMA,
        ]

    @functools.partial(
        pl.kernel,
        mesh=mesh,
        out_type=jax.ShapeDtypeStruct((B, D), jnp.float32),
        scratch_types=scratch,
        compiler_params=pltpu.CompilerParams(use_tc_tiling_on_sc=False),
    )
    def gather_kernel(idx_hbm, table_hbm, out_hbm, *bufs):
        rings = [tuple(bufs[4 * b : 4 * b + 4]) for b in range(_NBUF)]
        wid = lax.axis_index("s") * nc + lax.axis_index("c")
        base = wid * b_per_w

        def load_and_gather(g, b):
            idx_v, rows_v, gsem, _ = rings[b]
            pltpu.sync_copy(idx_hbm.at[pl.ds(base + g * _CHUNK, _CHUNK)], idx_v)
            pltpu.make_async_copy(table_hbm.at[idx_v], rows_v, gsem).start()

        for b in range(_NBUF):
            load_and_gather(b, b)

        @pl.loop(0, n_chunks, step=_NBUF)
        def _body(i):
            for b in range(_NBUF):
                idx_v, rows_v, gsem, ssem = rings[b]
                g = i + b
                dst = out_hbm.at[pl.ds(base + g * _CHUNK, _CHUNK)]
                pltpu.make_async_copy(table_hbm.at[idx_v], rows_v, gsem).wait()
                pltpu.make_async_copy(rows_v, dst, ssem).start()

                @pl.when(g + _NBUF < n_chunks)
                def _():
                    idx_v2, rows_v2, _, ssem2 = rings[b]
                    pltpu.sync_copy(
                        idx_hbm.at[pl.ds(base + (g + _NBUF) * _CHUNK, _CHUNK)],
                        idx_v2,
                    )
                    pltpu.make_async_copy(rows_v2, dst, ssem2).wait()
                    load_and_gather(g + _NBUF, b)

        for b in range(_NBUF):
            _, rows_v, _, ssem = rings[b]
            pltpu.make_async_copy(
                rows_v, out_hbm.at[pl.ds(0, _CHUNK)], ssem
            ).wait()

    return gather_kernel


def kernel(indices, weight):
    n, s = indices.shape
    d = weight.shape[1]
    flat_idx = indices.reshape(n * s).astype(jnp.int32)
    out = _make_gather(n * s, d)(flat_idx, weight)
    return out.reshape(n, s, d)


# speedup vs baseline: 1.8836x; 1.6998x over previous
import functools

import jax
import jax.numpy as jnp
from jax import lax
from jax.experimental import pallas as pl
from jax.experimental.pallas import tpu as pltpu
from jax.experimental.pallas import tpu_sc as plsc

_CHUNK = 1024


@functools.lru_cache(maxsize=None)
def _make_gather(N, S, D):
    info = plsc.get_sparse_core_info()
    nc, ns = info.num_cores, info.num_subcores
    nw = nc * ns
    chunks_per_s = N // _CHUNK
    n_chunks = S * chunks_per_s
    c_per_w = n_chunks // nw
    mesh = plsc.VectorSubcoreMesh(core_axis_name="c", subcore_axis_name="s")

    @functools.partial(
        pl.kernel,
        mesh=mesh,
        out_type=jax.ShapeDtypeStruct((S, N, D), jnp.float32),
        scratch_types=[
            pltpu.VMEM((_CHUNK,), jnp.int32),
            pltpu.VMEM((_CHUNK, D), jnp.float32),
            pltpu.SemaphoreType.DMA,
        ],
        compiler_params=pltpu.CompilerParams(use_tc_tiling_on_sc=False),
    )
    def gather_kernel(idx_hbm, table_hbm, out_hbm, idx_v, rows_v, sem):
        wid = lax.axis_index("s") * nc + lax.axis_index("c")
        c_base = wid * c_per_w

        def body(i, carry):
            c = c_base + i
            s = c // chunks_per_s
            b0 = (c % chunks_per_s) * _CHUNK
            pltpu.sync_copy(idx_hbm.at[pl.ds(c * _CHUNK, _CHUNK)], idx_v)
            pltpu.async_copy(table_hbm.at[idx_v], rows_v, sem).wait()
            pltpu.sync_copy(rows_v, out_hbm.at[s, pl.ds(b0, _CHUNK), :])
            return carry

        lax.fori_loop(0, c_per_w, body, 0)

    return gather_kernel


def kernel(indices, weight):
    n, s = indices.shape
    d = weight.shape[1]
    flat_idx = indices.T.reshape(s * n)
    out = _make_gather(n, s, d)(flat_idx, weight)
    return out.transpose(1, 0, 2)
